# speedup vs baseline: 1.6649x; 1.6649x over previous
import functools

import jax
import jax.numpy as jnp
from jax import lax
from jax.experimental import pallas as pl
from jax.experimental.pallas import tpu as pltpu
from jax.experimental.pallas import tpu_sc as plsc

VOCAB = 100000
HIDDEN = 2048
B = 8192

NUM_CORES = 2
NUM_SUBCORES = 16
NW = NUM_CORES * NUM_SUBCORES
BPW = B // NW
CHUNK = 16
NCHUNK = BPW // CHUNK


def _emb_kernel(idx_hbm, table_hbm, out_hbm, idx_v, rows_v, gsem):
    wid = lax.axis_index("s") * NUM_CORES + lax.axis_index("c")
    base = wid * BPW
    pltpu.sync_copy(idx_hbm.at[pl.ds(base, BPW)], idx_v)

    gathers = [None] * NCHUNK

    def issue(ch):
        return pltpu.async_copy(
            table_hbm.at[idx_v.at[pl.ds(ch * CHUNK, CHUNK)]],
            rows_v.at[ch % 2],
            gsem,
        )

    gathers[0] = issue(0)
    for ch in range(NCHUNK):
        if ch + 1 < NCHUNK:
            gathers[ch + 1] = issue(ch + 1)
        gathers[ch].wait()
        pltpu.sync_copy(rows_v.at[ch % 2],
                        out_hbm.at[pl.ds(base + ch * CHUNK, CHUNK)])


@jax.jit
def _emb(idx_flat, table):
    mesh = plsc.VectorSubcoreMesh(core_axis_name="c", subcore_axis_name="s")
    f = functools.partial(
        pl.kernel,
        mesh=mesh,
        out_type=jax.ShapeDtypeStruct((B, HIDDEN), jnp.float32),
        scratch_types=[
            pltpu.VMEM((BPW,), jnp.int32),
            pltpu.VMEM((2, CHUNK, HIDDEN), jnp.float32),
            pltpu.SemaphoreType.DMA,
        ],
    )(_emb_kernel)
    return f(idx_flat, table)


def kernel(token_ids, embed_weight):
    batch, seq = token_ids.shape
    idx_flat = token_ids.reshape(-1).astype(jnp.int32)
    out = _emb(idx_flat, embed_weight)
    return out.reshape(batch, seq, HIDDEN)
